# baseline (speedup 1.0000x reference)
import functools

import jax
import jax.numpy as jnp
from jax import lax
from jax.experimental import pallas as pl
from jax.experimental.pallas import tpu as pltpu
from jax.experimental.pallas import tpu_sc as plsc

B = 4096
DEG = 50
DEGP = 56
D = 64
NC = 2
NS = 16
NW = NC * NS
IDX_W = 128
ROWS_PER_W = (B * DEGP) // NW // IDX_W
EPW = ROWS_PER_W * IDX_W
NBUF = 3
GROUP = 4
GROWS = GROUP * IDX_W
NG = ROWS_PER_W // GROUP
UPW = B // NW

BB = 256
NBLK = BB * DEGP


def _sc_gather(item_emb, idx2, user_emb, nodes2):
    rows_per_w = idx2.shape[0] // NW
    epw = rows_per_w * IDX_W
    ng = rows_per_w // GROUP
    upw = nodes2.shape[1]
    nb = NW * upw
    mesh = plsc.VectorSubcoreMesh(
        core_axis_name="c", subcore_axis_name="s",
        num_cores=NC, num_subcores=NS)

    @functools.partial(
        pl.kernel,
        out_type=(
            jax.ShapeDtypeStruct((epw * NW, D), jnp.float32),
            jax.ShapeDtypeStruct((nb, D), jnp.float32),
        ),
        mesh=mesh,
        compiler_params=pltpu.CompilerParams(use_tc_tiling_on_sc=False),
        scratch_types=(
            pltpu.VMEM((rows_per_w, IDX_W), jnp.int32),
            pltpu.VMEM((NBUF, GROWS, D), jnp.float32),
            pltpu.VMEM((upw,), jnp.int32),
            pltpu.VMEM((upw, D), jnp.float32),
            pltpu.SemaphoreType.DMA((NBUF,)),
            pltpu.SemaphoreType.DMA((NBUF,)),
            pltpu.SemaphoreType.DMA,
        ),
    )
    def k(item_hbm, idx_hbm, user_hbm, nodes_hbm, g_hbm, u_hbm,
          idx_v, bufs, uidx_v, urows_v, gsems, osems, usem):
        wid = lax.axis_index("s") * NC + lax.axis_index("c")
        pltpu.sync_copy(idx_hbm.at[pl.ds(wid * rows_per_w, rows_per_w)], idx_v)
        pltpu.sync_copy(nodes_hbm.at[wid], uidx_v)
        ucp = pltpu.async_copy(user_hbm.at[uidx_v], urows_v, usem)
        obase = wid * epw

        def fire_g(g, b):
            for j in range(GROUP):
                pltpu.async_copy(
                    item_hbm.at[idx_v.at[g * GROUP + j]],
                    bufs.at[b, pl.ds(j * IDX_W, IDX_W)], gsems.at[b])

        def drain_g(b):
            pltpu.make_async_copy(
                item_hbm.at[pl.ds(0, GROWS)], bufs.at[b], gsems.at[b]).wait()

        def fire_o(g, b):
            pltpu.async_copy(
                bufs.at[b], g_hbm.at[pl.ds(obase + g * GROWS, GROWS)],
                osems.at[b])

        def drain_o(b):
            pltpu.make_async_copy(
                bufs.at[b], g_hbm.at[pl.ds(obase, GROWS)], osems.at[b]).wait()

        fire_g(0, 0)
        fire_g(1, 1)
        drain_g(0)
        fire_o(0, 0)
        fire_g(2, 2)

        def body(g, carry):
            b = lax.rem(g, NBUF)
            bp = lax.rem(g + 2, NBUF)
            drain_g(b)
            fire_o(g, b)
            drain_o(bp)
            fire_g(g + 2, bp)
            return carry

        lax.fori_loop(1, ng - 2, body, 0)
        drain_g((ng - 2) % NBUF)
        fire_o(ng - 2, (ng - 2) % NBUF)
        drain_o((ng - 3) % NBUF)
        drain_g((ng - 1) % NBUF)
        fire_o(ng - 1, (ng - 1) % NBUF)
        drain_o((ng - 2) % NBUF)
        drain_o((ng - 1) % NBUF)
        ucp.wait()
        pltpu.sync_copy(urows_v, u_hbm.at[pl.ds(wid * upw, upw)])

    return k(item_emb, idx2, user_emb, nodes2)


DP = DEGP // 2
N2BLK = BB * DP


def _tc_body(gp_ref, ids_ref, u_ref, w1blk_ref, w2blk_ref, wa1blk_ref,
             wa2blk_ref, wa3blk_ref, wa1bh_ref, w1rh_ref, remb_ref,
             b1p_ref, b2p_ref, ba1p_ref, ba2p_ref, out_ref):
    f32 = jnp.float32
    D2 = 2 * D
    gp = gp_ref[...]
    ids = ids_ref[...]
    r1 = jnp.dot(remb_ref[...], w1rh_ref[...],
                 preferred_element_type=f32)
    r1p = jnp.concatenate([r1, r1], axis=1)

    lane = lax.broadcasted_iota(jnp.int32, (N2BLK, D2), 1)
    idsx = jnp.where(lane < D, ids[:, 0:1], ids[:, 1:2])
    b0 = lax.bitwise_and(idsx, 1) == 1
    b1 = lax.bitwise_and(idsx, 2) == 2
    b2 = idsx >= 4
    a01 = jnp.where(b0, r1p[1:2, :], r1p[0:1, :])
    a23 = jnp.where(b0, r1p[3:4, :], r1p[2:3, :])
    rc = jnp.where(b2, r1p[4:5, :], jnp.where(b1, a23, a01))

    t = jnp.maximum(jnp.dot(gp, w1blk_ref[...], preferred_element_type=f32)
                    + rc + b1p_ref[...], 0.0)
    uvr = jnp.maximum(jnp.dot(t, w2blk_ref[...], preferred_element_type=f32)
                      + b2p_ref[...], 0.0)

    selfc = jnp.dot(u_ref[...], wa1bh_ref[...],
                    preferred_element_type=f32)
    selfp = jnp.concatenate([selfc, selfc], axis=1)
    h1 = jnp.dot(uvr, wa1blk_ref[...], preferred_element_type=f32)
    h = jnp.maximum(h1.reshape(BB, DP, D2) + selfp[:, None, :]
                    + ba1p_ref[...][None, :, :], 0.0)
    h2 = jnp.maximum(
        jnp.dot(h.reshape(N2BLK, D2), wa2blk_ref[...],
                preferred_element_type=f32) + ba2p_ref[...], 0.0)
    lg = jnp.dot(h2, wa3blk_ref[...], preferred_element_type=f32)

    l3 = lg.reshape(BB, DP, 2)
    ki = lax.broadcasted_iota(jnp.int32, (BB, DP, 2), 1)
    hi = lax.broadcasted_iota(jnp.int32, (BB, DP, 2), 2)
    l3 = jnp.where(2 * ki + hi < DEG, l3, f32(-1e30))
    m = jnp.max(jnp.max(l3, axis=2, keepdims=True), axis=1, keepdims=True)
    e = jnp.exp(l3 - m)
    se = jnp.sum(e, axis=1)
    s = se[:, 0:1] + se[:, 1:2]
    lane3 = lax.broadcasted_iota(jnp.int32, (BB, DP, D2), 2)
    esel = jnp.where(lane3 < D, e[:, :, 0:1], e[:, :, 1:2])
    nump = jnp.sum(uvr.reshape(BB, DP, D2) * esel, axis=1)
    out_ref[...] = (nump[:, 0:D] + nump[:, D:D2]) / s


def _tc_mlp(gp, ids2, u, w1blk, w2blk, wa1blk, wa2blk, wa3blk, wa1bh, w1rh,
            remb, b1p, b2p, ba1p, ba2p):
    D2 = 2 * D
    nb = u.shape[0]
    full = lambda shape: pl.BlockSpec(shape, lambda i: (0, 0))
    return pl.pallas_call(
        _tc_body,
        grid=(nb // BB,),
        in_specs=[
            pl.BlockSpec((N2BLK, D2), lambda i: (i, 0)),
            pl.BlockSpec((N2BLK, 2), lambda i: (i, 0)),
            pl.BlockSpec((BB, D), lambda i: (i, 0)),
            full((D2, D2)), full((D2, D2)), full((D2, D2)), full((D2, D2)),
            full((D2, 2)), full((D, D)), full((D, D)), full((8, D)),
            full((1, D2)), full((1, D2)), full((1, D2)), full((1, D2)),
        ],
        out_specs=pl.BlockSpec((BB, D), lambda i: (i, 0)),
        out_shape=jax.ShapeDtypeStruct((nb, D), jnp.float32),
    )(gp, ids2, u, w1blk, w2blk, wa1blk, wa2blk, wa3blk, wa1bh, w1rh,
      remb, b1p, b2p, ba1p, ba2p)


def _blkdiag(w):
    z = jnp.zeros_like(w)
    return jnp.concatenate(
        [jnp.concatenate([w, z], axis=1), jnp.concatenate([z, w], axis=1)],
        axis=0)


def kernel(nodes, uv_adjacency, ratings, user_emb, item_emb, rating_emb,
           w1_w, w1_b, w2_w, w2_b, wa1_w, wa1_b, wa2_w, wa2_b, wa3_w, wa3_b):
    npad = DEGP - DEG
    pad_idx = (lax.broadcasted_iota(jnp.int32, (B, npad), 0) * npad
               + lax.broadcasted_iota(jnp.int32, (B, npad), 1))
    adj_p = jnp.concatenate([uv_adjacency.astype(jnp.int32), pad_idx], axis=1)
    rat_p = jnp.pad(ratings.astype(jnp.int32), ((0, 0), (0, npad)))
    nodes_i = nodes.astype(jnp.int32)

    remb = jnp.pad(rating_emb, ((0, 3), (0, 0)))
    pair = lambda v: jnp.concatenate([v, v], axis=0).reshape(1, 2 * D)
    weights = (
        _blkdiag(w1_w[:D]), _blkdiag(w2_w), _blkdiag(wa1_w[:D]),
        _blkdiag(wa2_w),
        _blkdiag(wa3_w),
        wa1_w[D:], w1_w[D:], remb,
        pair(w1_b), pair(w2_b), pair(wa1_b), pair(wa2_b))

    H = B // 2
    outs = []
    prev_u = None
    for h in range(2):
        adj_h = lax.slice_in_dim(adj_p, h * H, (h + 1) * H, axis=0)
        idx2 = adj_h.reshape(H * DEGP // IDX_W, IDX_W)
        nodes2 = lax.slice_in_dim(nodes_i, h * H, (h + 1) * H).reshape(
            NW, H // NW)
        if prev_u is not None:
            idx2, nodes2, _ = lax.optimization_barrier((idx2, nodes2, prev_u))
        g, u = _sc_gather(item_emb, idx2, user_emb, nodes2)
        prev_u = u
        gp = g.reshape(H * DEGP // 2, 2 * D)
        ids2 = lax.slice_in_dim(rat_p, h * H, (h + 1) * H, axis=0).reshape(
            H * DEGP // 2, 2)
        outs.append(_tc_mlp(gp, ids2, u, *weights))
    return jnp.concatenate(outs, axis=0)


# speedup vs baseline: 1.0437x; 1.0437x over previous
import functools

import jax
import jax.numpy as jnp
from jax import lax
from jax.experimental import pallas as pl
from jax.experimental.pallas import tpu as pltpu
from jax.experimental.pallas import tpu_sc as plsc

B = 4096
DEG = 50
DEGP = 56
D = 64
NC = 2
NS = 16
NW = NC * NS
IDX_W = 128
ROWS_PER_W = (B * DEGP) // NW // IDX_W
EPW = ROWS_PER_W * IDX_W
NBUF = 3
GROUP = 4
GROWS = GROUP * IDX_W
NG = ROWS_PER_W // GROUP
UPW = B // NW

BB = 512
NBLK = BB * DEGP


def _sc_gather(item_emb, idx2, user_emb, nodes2):
    rows_per_w = idx2.shape[0] // NW
    epw = rows_per_w * IDX_W
    ng = rows_per_w // GROUP
    upw = nodes2.shape[1]
    nb = NW * upw
    mesh = plsc.VectorSubcoreMesh(
        core_axis_name="c", subcore_axis_name="s",
        num_cores=NC, num_subcores=NS)

    @functools.partial(
        pl.kernel,
        out_type=(
            jax.ShapeDtypeStruct((epw * NW, D), jnp.float32),
            jax.ShapeDtypeStruct((nb, D), jnp.float32),
        ),
        mesh=mesh,
        compiler_params=pltpu.CompilerParams(use_tc_tiling_on_sc=False),
        scratch_types=(
            pltpu.VMEM((rows_per_w, IDX_W), jnp.int32),
            pltpu.VMEM((NBUF, GROWS, D), jnp.float32),
            pltpu.VMEM((upw,), jnp.int32),
            pltpu.VMEM((upw, D), jnp.float32),
            pltpu.SemaphoreType.DMA((NBUF,)),
            pltpu.SemaphoreType.DMA((NBUF,)),
            pltpu.SemaphoreType.DMA,
        ),
    )
    def k(item_hbm, idx_hbm, user_hbm, nodes_hbm, g_hbm, u_hbm,
          idx_v, bufs, uidx_v, urows_v, gsems, osems, usem):
        wid = lax.axis_index("s") * NC + lax.axis_index("c")
        pltpu.sync_copy(idx_hbm.at[pl.ds(wid * rows_per_w, rows_per_w)], idx_v)
        pltpu.sync_copy(nodes_hbm.at[wid], uidx_v)
        ucp = pltpu.async_copy(user_hbm.at[uidx_v], urows_v, usem)
        obase = wid * epw

        def fire_g(g, b):
            for j in range(GROUP):
                pltpu.async_copy(
                    item_hbm.at[idx_v.at[g * GROUP + j]],
                    bufs.at[b, pl.ds(j * IDX_W, IDX_W)], gsems.at[b])

        def drain_g(b):
            pltpu.make_async_copy(
                item_hbm.at[pl.ds(0, GROWS)], bufs.at[b], gsems.at[b]).wait()

        def fire_o(g, b):
            pltpu.async_copy(
                bufs.at[b], g_hbm.at[pl.ds(obase + g * GROWS, GROWS)],
                osems.at[b])

        def drain_o(b):
            pltpu.make_async_copy(
                bufs.at[b], g_hbm.at[pl.ds(obase, GROWS)], osems.at[b]).wait()

        fire_g(0, 0)
        fire_g(1, 1)
        drain_g(0)
        fire_o(0, 0)
        fire_g(2, 2)

        def body(g, carry):
            b = lax.rem(g, NBUF)
            bp = lax.rem(g + 2, NBUF)
            drain_g(b)
            fire_o(g, b)
            drain_o(bp)
            fire_g(g + 2, bp)
            return carry

        lax.fori_loop(1, ng - 2, body, 0)
        drain_g((ng - 2) % NBUF)
        fire_o(ng - 2, (ng - 2) % NBUF)
        drain_o((ng - 3) % NBUF)
        drain_g((ng - 1) % NBUF)
        fire_o(ng - 1, (ng - 1) % NBUF)
        drain_o((ng - 2) % NBUF)
        drain_o((ng - 1) % NBUF)
        ucp.wait()
        pltpu.sync_copy(urows_v, u_hbm.at[pl.ds(wid * upw, upw)])

    return k(item_emb, idx2, user_emb, nodes2)


DP = DEGP // 2
N2BLK = BB * DP


def _tc_body(gp_ref, ids_ref, u_ref, w1blk_ref, w2blk_ref, wa1blk_ref,
             wa2blk_ref, wa3blk_ref, wa1bh_ref, w1rh_ref, remb_ref,
             b1p_ref, b2p_ref, ba1p_ref, ba2p_ref, out_ref):
    f32 = jnp.float32
    D2 = 2 * D
    gp = gp_ref[...]
    ids = ids_ref[...]
    r1 = jnp.dot(remb_ref[...], w1rh_ref[...],
                 preferred_element_type=f32)
    r1p = jnp.concatenate([r1, r1], axis=1)

    lane = lax.broadcasted_iota(jnp.int32, (N2BLK, D2), 1)
    idsx = jnp.where(lane < D, ids[:, 0:1], ids[:, 1:2])
    b0 = lax.bitwise_and(idsx, 1) == 1
    b1 = lax.bitwise_and(idsx, 2) == 2
    b2 = idsx >= 4
    a01 = jnp.where(b0, r1p[1:2, :], r1p[0:1, :])
    a23 = jnp.where(b0, r1p[3:4, :], r1p[2:3, :])
    rc = jnp.where(b2, r1p[4:5, :], jnp.where(b1, a23, a01))

    t = jnp.maximum(jnp.dot(gp, w1blk_ref[...], preferred_element_type=f32)
                    + rc + b1p_ref[...], 0.0)
    uvr = jnp.maximum(jnp.dot(t, w2blk_ref[...], preferred_element_type=f32)
                      + b2p_ref[...], 0.0)

    selfc = jnp.dot(u_ref[...], wa1bh_ref[...],
                    preferred_element_type=f32)
    selfp = jnp.concatenate([selfc, selfc], axis=1)
    h1 = jnp.dot(uvr, wa1blk_ref[...], preferred_element_type=f32)
    h = jnp.maximum(h1.reshape(BB, DP, D2) + selfp[:, None, :]
                    + ba1p_ref[...][None, :, :], 0.0)
    h2 = jnp.maximum(
        jnp.dot(h.reshape(N2BLK, D2), wa2blk_ref[...],
                preferred_element_type=f32) + ba2p_ref[...], 0.0)
    lg = jnp.dot(h2, wa3blk_ref[...], preferred_element_type=f32)

    l3 = lg.reshape(BB, DP, 2)
    ki = lax.broadcasted_iota(jnp.int32, (BB, DP, 2), 1)
    hi = lax.broadcasted_iota(jnp.int32, (BB, DP, 2), 2)
    l3 = jnp.where(2 * ki + hi < DEG, l3, f32(-1e30))
    m = jnp.max(jnp.max(l3, axis=2, keepdims=True), axis=1, keepdims=True)
    e = jnp.exp(l3 - m)
    se = jnp.sum(e, axis=1)
    s = se[:, 0:1] + se[:, 1:2]
    lane3 = lax.broadcasted_iota(jnp.int32, (BB, DP, D2), 2)
    esel = jnp.where(lane3 < D, e[:, :, 0:1], e[:, :, 1:2])
    nump = jnp.sum(uvr.reshape(BB, DP, D2) * esel, axis=1)
    out_ref[...] = (nump[:, 0:D] + nump[:, D:D2]) / s


def _tc_mlp(gp, ids2, u, w1blk, w2blk, wa1blk, wa2blk, wa3blk, wa1bh, w1rh,
            remb, b1p, b2p, ba1p, ba2p):
    D2 = 2 * D
    nb = u.shape[0]
    full = lambda shape: pl.BlockSpec(shape, lambda i: (0, 0))
    return pl.pallas_call(
        _tc_body,
        grid=(nb // BB,),
        in_specs=[
            pl.BlockSpec((N2BLK, D2), lambda i: (i, 0)),
            pl.BlockSpec((N2BLK, 2), lambda i: (i, 0)),
            pl.BlockSpec((BB, D), lambda i: (i, 0)),
            full((D2, D2)), full((D2, D2)), full((D2, D2)), full((D2, D2)),
            full((D2, 2)), full((D, D)), full((D, D)), full((8, D)),
            full((1, D2)), full((1, D2)), full((1, D2)), full((1, D2)),
        ],
        out_specs=pl.BlockSpec((BB, D), lambda i: (i, 0)),
        out_shape=jax.ShapeDtypeStruct((nb, D), jnp.float32),
        compiler_params=pltpu.CompilerParams(
            vmem_limit_bytes=100 * 1024 * 1024),
    )(gp, ids2, u, w1blk, w2blk, wa1blk, wa2blk, wa3blk, wa1bh, w1rh,
      remb, b1p, b2p, ba1p, ba2p)


def _blkdiag(w):
    z = jnp.zeros_like(w)
    return jnp.concatenate(
        [jnp.concatenate([w, z], axis=1), jnp.concatenate([z, w], axis=1)],
        axis=0)


def kernel(nodes, uv_adjacency, ratings, user_emb, item_emb, rating_emb,
           w1_w, w1_b, w2_w, w2_b, wa1_w, wa1_b, wa2_w, wa2_b, wa3_w, wa3_b):
    npad = DEGP - DEG
    pad_idx = (lax.broadcasted_iota(jnp.int32, (B, npad), 0) * npad
               + lax.broadcasted_iota(jnp.int32, (B, npad), 1))
    adj_p = jnp.concatenate([uv_adjacency.astype(jnp.int32), pad_idx], axis=1)
    rat_p = jnp.pad(ratings.astype(jnp.int32), ((0, 0), (0, npad)))
    nodes_i = nodes.astype(jnp.int32)

    remb = jnp.pad(rating_emb, ((0, 3), (0, 0)))
    pair = lambda v: jnp.concatenate([v, v], axis=0).reshape(1, 2 * D)
    weights = (
        _blkdiag(w1_w[:D]), _blkdiag(w2_w), _blkdiag(wa1_w[:D]),
        _blkdiag(wa2_w),
        _blkdiag(wa3_w),
        wa1_w[D:], w1_w[D:], remb,
        pair(w1_b), pair(w2_b), pair(wa1_b), pair(wa2_b))

    idx2 = adj_p.reshape(B * DEGP // IDX_W, IDX_W)
    nodes2 = nodes_i.reshape(NW, B // NW)
    g, u = _sc_gather(item_emb, idx2, user_emb, nodes2)
    gp = g.reshape(B * DEGP // 2, 2 * D)
    ids2 = rat_p.reshape(B * DEGP // 2, 2)
    return _tc_mlp(gp, ids2, u, *weights)
